# baseline (speedup 1.0000x reference)
import functools

import jax
import jax.numpy as jnp
from jax import lax
from jax.experimental import pallas as pl
from jax.experimental.pallas import tpu as pltpu
from jax.experimental.pallas import tpu_sc as plsc

_NC = 2
_NS = 16
_NW = _NC * _NS
_C = 16


def kernel(inputs, pos_table):
    B, S, D = inputs.shape
    rows_w = S // _NW
    n_chunks = rows_w // _C
    n_iters = n_chunks // 2

    mesh = plsc.VectorSubcoreMesh(core_axis_name="c", subcore_axis_name="s")

    @functools.partial(
        pl.kernel,
        out_type=jax.ShapeDtypeStruct((B, S, D), jnp.float32),
        mesh=mesh,
        scratch_types=[
            pltpu.VMEM((2, B, _C, D), jnp.float32),
            pltpu.VMEM((2, _C, D), jnp.float32),
        ]
        + [pltpu.SemaphoreType.DMA] * 6,
    )
    def sc_add(x_hbm, p_hbm, o_hbm, xv, pv, sx0, sx1, so0, so1, sp0, sp1):
        sx = (sx0, sx1)
        so = (so0, so1)
        sp = (sp0, sp1)
        wid = lax.axis_index("s") * _NC + lax.axis_index("c")
        row0 = wid * rows_w

        def start_x(cidx, slot):
            return pltpu.async_copy(
                x_hbm.at[:, pl.ds(row0 + cidx * _C, _C), :], xv.at[slot], sx[slot]
            )

        def start_p(cidx, slot):
            return pltpu.async_copy(
                p_hbm.at[pl.ds(row0 + cidx * _C, _C), :], pv.at[slot], sp[slot]
            )

        def start_out(cidx, slot):
            return pltpu.async_copy(
                xv.at[slot], o_hbm.at[:, pl.ds(row0 + cidx * _C, _C), :], so[slot]
            )

        def wait_x(sem):
            pltpu.make_async_copy(
                x_hbm.at[:, pl.ds(row0, _C), :], xv.at[0], sem
            ).wait()

        def wait_p(sem):
            pltpu.make_async_copy(
                p_hbm.at[pl.ds(row0, _C), :], pv.at[0], sem
            ).wait()

        def compute(slot):
            @plsc.parallel_loop(0, _C)
            def row_body(r):
                pr = pv.at[slot, r]
                xr = [xv.at[slot, b, r] for b in range(B)]
                for j in range(D // 16):
                    sl = pl.ds(j * 16, 16)
                    pj = pr[sl]
                    for b in range(B):
                        xr[b][sl] = xr[b][sl] + pj

        start_p(0, 0)
        start_x(0, 0)

        def outer(cc, _):
            for u in range(2):
                k = u
                cidx = 2 * cc + u
                wait_p(sp[k])
                wait_x(sx[k])
                compute(k)
                start_out(cidx, k)
                if u == 0:
                    @pl.when(cc > 0)
                    def _():
                        wait_x(so[1])

                    start_p(2 * cc + 1, 1)
                    start_x(2 * cc + 1, 1)
                else:
                    wait_x(so[0])

                    @pl.when(cc < n_iters - 1)
                    def _():
                        start_p(2 * cc + 2, 0)
                        start_x(2 * cc + 2, 0)
            return 0

        lax.fori_loop(0, n_iters, outer, 0)
        wait_x(so[1])

    return sc_add(inputs, pos_table)


# speedup vs baseline: 1.1604x; 1.1604x over previous
import functools

import jax
import jax.numpy as jnp
from jax import lax
from jax.experimental import pallas as pl
from jax.experimental.pallas import tpu as pltpu
from jax.experimental.pallas import tpu_sc as plsc

_NC = 2
_NS = 16
_NW = _NC * _NS
_C = 16
_NB = 8


def kernel(inputs, pos_table):
    B, S, D = inputs.shape
    rows_w = S // _NW
    n_chunks = rows_w // _C
    n_units = n_chunks * B
    units_per_iter = 2 * B
    n_iters = n_units // units_per_iter

    mesh = plsc.VectorSubcoreMesh(core_axis_name="c", subcore_axis_name="s")

    @functools.partial(
        pl.kernel,
        out_type=jax.ShapeDtypeStruct((B, S, D), jnp.float32),
        mesh=mesh,
        scratch_types=[
            pltpu.VMEM((_NB, _C, D), jnp.float32),
            pltpu.VMEM((2, _C, D), jnp.float32),
        ]
        + [pltpu.SemaphoreType.DMA] * (_NB + _NB + 2),
    )
    def sc_add(x_hbm, p_hbm, o_hbm, xv, pv, *sems):
        sx = sems[:_NB]
        so = sems[_NB:2 * _NB]
        sp = sems[2 * _NB:]
        wid = lax.axis_index("s") * _NC + lax.axis_index("c")
        row0 = wid * rows_w

        def start_x(cidx, b, slot):
            return pltpu.async_copy(
                x_hbm.at[b, pl.ds(row0 + cidx * _C, _C), :], xv.at[slot], sx[slot]
            )

        def start_p(cidx, pslot):
            return pltpu.async_copy(
                p_hbm.at[pl.ds(row0 + cidx * _C, _C), :], pv.at[pslot], sp[pslot]
            )

        def start_out(cidx, b, slot):
            return pltpu.async_copy(
                xv.at[slot], o_hbm.at[b, pl.ds(row0 + cidx * _C, _C), :], so[slot]
            )

        def wait_chunk(sem):
            pltpu.make_async_copy(
                x_hbm.at[0, pl.ds(row0, _C), :], xv.at[0], sem
            ).wait()

        def compute(xslot, pslot):
            @plsc.parallel_loop(0, _C)
            def row_body(r):
                xr = xv.at[xslot, r]
                pr = pv.at[pslot, r]
                for j in range(D // 16):
                    sl = pl.ds(j * 16, 16)
                    xr[sl] = xr[sl] + pr[sl]

        start_p(0, 0)
        for g in range(_NB - 1):
            start_x(g // B, g % B, g % _NB)

        def outer(cc, _):
            for u in range(units_per_iter):
                half = u // B
                b = u % B
                cidx = 2 * cc + half
                xslot = u % _NB
                pslot = half
                if u == 0:
                    wait_chunk(sp[0])
                    start_p(2 * cc + 1, 1)
                if u == B:
                    wait_chunk(sp[1])

                    @pl.when(cc < n_iters - 1)
                    def _():
                        start_p(2 * cc + 2, 0)

                wait_chunk(sx[xslot])
                compute(xslot, pslot)
                start_out(cidx, b, xslot)
                if u == 0:
                    @pl.when(cc > 0)
                    def _():
                        wait_chunk(so[(u + _NB - 1) % _NB])
                else:
                    wait_chunk(so[(u + _NB - 1) % _NB])
                t = u + _NB - 1
                if t < units_per_iter:
                    start_x(2 * cc + t // B, t % B, t % _NB)
                else:
                    tn = t - units_per_iter

                    @pl.when(cc < n_iters - 1)
                    def _():
                        start_x(2 * (cc + 1) + tn // B, tn % B, t % _NB)
            return 0

        lax.fori_loop(0, n_iters, outer, 0)
        wait_chunk(so[(n_units - 1) % _NB])

    return sc_add(inputs, pos_table)


# speedup vs baseline: 1.1797x; 1.0167x over previous
import functools

import jax
import jax.numpy as jnp
from jax import lax
from jax.experimental import pallas as pl
from jax.experimental.pallas import tpu as pltpu
from jax.experimental.pallas import tpu_sc as plsc

_NC = 2
_NS = 16
_NW = _NC * _NS
_C = 16
_NB = 4


def kernel(inputs, pos_table):
    B, S, D = inputs.shape
    rows_w = S // _NW
    n_chunks = rows_w // _C
    n_units = n_chunks * B
    units_per_iter = 2 * B
    n_iters = n_units // units_per_iter

    mesh = plsc.VectorSubcoreMesh(core_axis_name="c", subcore_axis_name="s")

    @functools.partial(
        pl.kernel,
        out_type=jax.ShapeDtypeStruct((B, S, D), jnp.float32),
        mesh=mesh,
        scratch_types=[
            pltpu.VMEM((_NB, _C, D), jnp.float32),
            pltpu.VMEM((2, _C, D), jnp.float32),
        ]
        + [pltpu.SemaphoreType.DMA] * (_NB + _NB + 2),
    )
    def sc_add(x_hbm, p_hbm, o_hbm, xv, pv, *sems):
        sx = sems[:_NB]
        so = sems[_NB:2 * _NB]
        sp = sems[2 * _NB:]
        wid = lax.axis_index("s") * _NC + lax.axis_index("c")
        row0 = wid * rows_w

        def start_x(cidx, b, slot):
            return pltpu.async_copy(
                x_hbm.at[b, pl.ds(row0 + cidx * _C, _C), :], xv.at[slot], sx[slot]
            )

        def start_p(cidx, pslot):
            return pltpu.async_copy(
                p_hbm.at[pl.ds(row0 + cidx * _C, _C), :], pv.at[pslot], sp[pslot]
            )

        def start_out(cidx, b, slot):
            return pltpu.async_copy(
                xv.at[slot], o_hbm.at[b, pl.ds(row0 + cidx * _C, _C), :], so[slot]
            )

        def wait_chunk(sem):
            pltpu.make_async_copy(
                x_hbm.at[0, pl.ds(row0, _C), :], xv.at[0], sem
            ).wait()

        def compute(xslot, pslot):
            @plsc.parallel_loop(0, _C)
            def row_body(r):
                xr = xv.at[xslot, r]
                pr = pv.at[pslot, r]

                @plsc.parallel_loop(0, D // 128)
                def j_body(j):
                    for u in range(8):
                        sl = pl.ds(j * 128 + u * 16, 16)
                        xr[sl] = xr[sl] + pr[sl]

        start_p(0, 0)
        for g in range(_NB - 1):
            start_x(g // B, g % B, g % _NB)

        def outer(cc, _):
            for u in range(units_per_iter):
                half = u // B
                b = u % B
                cidx = 2 * cc + half
                xslot = u % _NB
                pslot = half
                if u == 0:
                    wait_chunk(sp[0])
                    start_p(2 * cc + 1, 1)
                if u == B:
                    wait_chunk(sp[1])

                    @pl.when(cc < n_iters - 1)
                    def _():
                        start_p(2 * cc + 2, 0)

                wait_chunk(sx[xslot])
                compute(xslot, pslot)
                start_out(cidx, b, xslot)
                if u == 0:
                    @pl.when(cc > 0)
                    def _():
                        wait_chunk(so[(u + 3) % _NB])
                else:
                    wait_chunk(so[(u + 3) % _NB])
                t = u + _NB - 1
                if t < units_per_iter:
                    start_x(2 * cc + t // B, t % B, t % _NB)
                else:
                    tn = t - units_per_iter

                    @pl.when(cc < n_iters - 1)
                    def _():
                        start_x(2 * (cc + 1) + tn // B, tn % B, t % _NB)
            return 0

        lax.fori_loop(0, n_iters, outer, 0)
        wait_chunk(so[(n_units - 1) % _NB])

    return sc_add(inputs, pos_table)


# speedup vs baseline: 1.2209x; 1.0349x over previous
import functools

import jax
import jax.numpy as jnp
from jax import lax
from jax.experimental import pallas as pl
from jax.experimental.pallas import tpu as pltpu
from jax.experimental.pallas import tpu_sc as plsc

_NC = 2
_NS = 16
_NW = _NC * _NS
_C = 8
_NB = 4


def kernel(inputs, pos_table):
    B, S, D = inputs.shape
    rows_w = S // _NW
    n_chunks = rows_w // _C
    n_units = n_chunks * B
    units_per_iter = 2 * B
    n_iters = n_units // units_per_iter

    mesh = plsc.VectorSubcoreMesh(core_axis_name="c", subcore_axis_name="s")

    @functools.partial(
        pl.kernel,
        out_type=jax.ShapeDtypeStruct((B, S, D), jnp.float32),
        mesh=mesh,
        scratch_types=[
            pltpu.VMEM((_NB, _C, D), jnp.float32),
            pltpu.VMEM((2, _C, D), jnp.float32),
        ]
        + [pltpu.SemaphoreType.DMA] * (_NB + _NB + 2),
    )
    def sc_add(x_hbm, p_hbm, o_hbm, xv, pv, *sems):
        sx = sems[:_NB]
        so = sems[_NB:2 * _NB]
        sp = sems[2 * _NB:]
        wid = lax.axis_index("s") * _NC + lax.axis_index("c")
        row0 = wid * rows_w

        def start_x(cidx, b, slot):
            return pltpu.async_copy(
                x_hbm.at[b, pl.ds(row0 + cidx * _C, _C), :], xv.at[slot], sx[slot]
            )

        def start_p(cidx, pslot):
            return pltpu.async_copy(
                p_hbm.at[pl.ds(row0 + cidx * _C, _C), :], pv.at[pslot], sp[pslot]
            )

        def start_out(cidx, b, slot):
            return pltpu.async_copy(
                xv.at[slot], o_hbm.at[b, pl.ds(row0 + cidx * _C, _C), :], so[slot]
            )

        def wait_chunk(sem):
            pltpu.make_async_copy(
                x_hbm.at[0, pl.ds(row0, _C), :], xv.at[0], sem
            ).wait()

        def compute(xslot, pslot):
            @plsc.parallel_loop(0, _C)
            def row_body(r):
                xr = xv.at[xslot, r]
                pr = pv.at[pslot, r]
                for j in range(D // 16):
                    sl = pl.ds(j * 16, 16)
                    xr[sl] = xr[sl] + pr[sl]

        start_p(0, 0)
        for g in range(_NB - 1):
            start_x(g // B, g % B, g % _NB)

        def outer(cc, _):
            for u in range(units_per_iter):
                half = u // B
                b = u % B
                cidx = 2 * cc + half
                xslot = u % _NB
                pslot = half
                if u == 0:
                    wait_chunk(sp[0])
                    start_p(2 * cc + 1, 1)
                if u == B:
                    wait_chunk(sp[1])

                    @pl.when(cc < n_iters - 1)
                    def _():
                        start_p(2 * cc + 2, 0)

                wait_chunk(sx[xslot])
                compute(xslot, pslot)
                start_out(cidx, b, xslot)
                if u == 0:
                    @pl.when(cc > 0)
                    def _():
                        wait_chunk(so[(u + 3) % _NB])
                else:
                    wait_chunk(so[(u + 3) % _NB])
                t = u + _NB - 1
                if t < units_per_iter:
                    start_x(2 * cc + t // B, t % B, t % _NB)
                else:
                    tn = t - units_per_iter

                    @pl.when(cc < n_iters - 1)
                    def _():
                        start_x(2 * (cc + 1) + tn // B, tn % B, t % _NB)
            return 0

        lax.fori_loop(0, n_iters, outer, 0)
        wait_chunk(so[(n_units - 1) % _NB])

    return sc_add(inputs, pos_table)


# speedup vs baseline: 1.4268x; 1.1686x over previous
import functools

import jax
import jax.numpy as jnp
from jax import lax
from jax.experimental import pallas as pl
from jax.experimental.pallas import tpu as pltpu
from jax.experimental.pallas import tpu_sc as plsc

_NC = 2
_NS = 16
_NW = _NC * _NS
_C = 16
_NB = 4


def kernel(inputs, pos_table):
    B, S, D = inputs.shape
    rows_w = S // _NW
    n_chunks = rows_w // _C
    n_units = n_chunks * B
    units_per_iter = 2 * B
    n_iters = n_units // units_per_iter

    mesh = plsc.VectorSubcoreMesh(core_axis_name="c", subcore_axis_name="s")

    @functools.partial(
        pl.kernel,
        out_type=jax.ShapeDtypeStruct((B, S, D), jnp.float32),
        mesh=mesh,
        scratch_types=[
            pltpu.VMEM((_NB, _C, D), jnp.float32),
            pltpu.VMEM((2, _C, D), jnp.float32),
        ]
        + [pltpu.SemaphoreType.DMA] * (_NB + _NB + 2),
    )
    def sc_add(x_hbm, p_hbm, o_hbm, xv, pv, *sems):
        sx = sems[:_NB]
        so = sems[_NB:2 * _NB]
        sp = sems[2 * _NB:]
        wid = lax.axis_index("s") * _NC + lax.axis_index("c")
        row0 = wid * rows_w

        def start_x(cidx, b, slot):
            return pltpu.async_copy(
                x_hbm.at[b, pl.ds(row0 + cidx * _C, _C), :], xv.at[slot], sx[slot]
            )

        def start_p(cidx, pslot):
            return pltpu.async_copy(
                p_hbm.at[pl.ds(row0 + cidx * _C, _C), :], pv.at[pslot], sp[pslot]
            )

        def start_out(cidx, b, slot):
            return pltpu.async_copy(
                xv.at[slot], o_hbm.at[b, pl.ds(row0 + cidx * _C, _C), :], so[slot]
            )

        def wait_chunk(sem):
            pltpu.make_async_copy(
                x_hbm.at[0, pl.ds(row0, _C), :], xv.at[0], sem
            ).wait()

        def compute(xslot, pslot):
            @plsc.parallel_loop(0, _C)
            def row_body(r):
                xr = xv.at[xslot, r]
                pr = pv.at[pslot, r]
                for j in range(D // 16):
                    sl = pl.ds(j * 16, 16)
                    xr[sl] = xr[sl] + pr[sl]

        start_p(0, 0)
        for g in range(_NB - 1):
            start_x(g // B, g % B, g % _NB)

        def outer(cc, _):
            for u in range(units_per_iter):
                half = u // B
                b = u % B
                cidx = 2 * cc + half
                xslot = u % _NB
                pslot = half
                if u == 0:
                    wait_chunk(sp[0])
                    start_p(2 * cc + 1, 1)
                if u == B:
                    wait_chunk(sp[1])

                    @pl.when(cc < n_iters - 1)
                    def _():
                        start_p(2 * cc + 2, 0)

                wait_chunk(sx[xslot])
                compute(xslot, pslot)
                start_out(cidx, b, xslot)
                if u == 0:
                    @pl.when(cc > 0)
                    def _():
                        wait_chunk(so[(u + 3) % _NB])
                else:
                    wait_chunk(so[(u + 3) % _NB])
                t = u + _NB - 1
                if t < units_per_iter:
                    start_x(2 * cc + t // B, t % B, t % _NB)
                else:
                    tn = t - units_per_iter

                    @pl.when(cc < n_iters - 1)
                    def _():
                        start_x(2 * (cc + 1) + tn // B, tn % B, t % _NB)
            return 0

        lax.fori_loop(0, n_iters, outer, 0)
        wait_chunk(so[(n_units - 1) % _NB])

    return sc_add(inputs, pos_table)
